# baseline (speedup 1.0000x reference)
import jax
import jax.numpy as jnp
from jax import lax
from jax.experimental import pallas as pl
from jax.experimental.pallas import tpu as pltpu
from jax.experimental.pallas import tpu_sc as plsc

_S = 768
_GL = 3
_NV = 256
_ROW = _S * _GL
_BU = 128
_BC = 384
_NJ = _S // _BC
_NK = _S // _BU


def _sc_body(a_hbm, out_hbm, x_v, outb_v):
    wid = lax.axis_index("c") * 16 + lax.axis_index("s")

    lane = lax.iota(jnp.int32, 16)
    u0 = (lane * 11) >> 5
    c0 = lane - 3 * u0

    def jk_loop(jk, _):
        j = jk // _NK
        k = jk - j * _NK
        pltpu.sync_copy(
            a_hbm.at[
                wid,
                pl.ds(pl.multiple_of(k * _BU, _BU), _BU),
                pl.ds(pl.multiple_of(j * _BC, _BC), _BC),
            ],
            x_v,
        )

        def col_loop(s, carry):
            u_idx, c_idx = carry
            for v in range(_BU):
                val = plsc.load_gather(x_v, [u_idx, c_idx + (_GL * v)])
                outb_v[v, pl.ds(s * 16, 16)] = val
            wrap = c_idx == 2
            u_n = u_idx + jnp.where(wrap, 6, 5)
            c_n = jnp.where(wrap, 0, c_idx + 1)
            return (u_n, c_n)

        lax.fori_loop(0, _BC // 16, col_loop, (u0, c0))
        pltpu.sync_copy(
            outb_v,
            out_hbm.at[
                wid,
                pl.ds(pl.multiple_of(j * _BU, _BU), _BU),
                pl.ds(pl.multiple_of(k * _BC, _BC), _BC),
            ],
        )
        return _

    lax.fori_loop(0, _NJ * _NK, jk_loop, 0)


def kernel(inputs):
    A = inputs
    B, H, S1, S2 = A.shape
    a = A.reshape(B * H, S1, S2)
    mesh = plsc.VectorSubcoreMesh(
        core_axis_name="c", subcore_axis_name="s", num_cores=2, num_subcores=16
    )
    f = pl.kernel(
        _sc_body,
        mesh=mesh,
        compiler_params=pltpu.CompilerParams(needs_layout_passes=False),
        out_type=jax.ShapeDtypeStruct((B * H, _NV, _ROW), jnp.float32),
        scratch_types=[
            pltpu.VMEM((_BU, _BC), jnp.float32),
            pltpu.VMEM((_BU, _BC), jnp.float32),
        ],
    )
    out = f(a)
    return out.reshape(B, H, S1 * S2 // (_GL * _GL), _GL, _GL)


# speedup vs baseline: 5.4981x; 5.4981x over previous
import jax
import jax.numpy as jnp
from jax import lax
from jax.experimental import pallas as pl
from jax.experimental.pallas import tpu as pltpu
from jax.experimental.pallas import tpu_sc as plsc

_S = 768
_GL = 3
_NT = 32
_CW = 24
_DV = 8
_NP = 9


def _sc_body(a_hbm, out_hbm, strip_v, outb_v):
    wid = lax.axis_index("c") * 16 + lax.axis_index("s")
    batch = wid // 16
    h = wid - batch * 16
    th = h // 8
    hh = h - th * 8

    lane = lax.iota(jnp.int32, 16)
    r3 = lane * 3

    def strip_loop(t, carry):
        pltpu.sync_copy(a_hbm.at[wid, :, pl.ds(t * _CW, _CW)], strip_v)

        def lane_loop(s, inner):
            ti_off = s >> 3
            c_off = (s & 7) * 16
            rbase = r3 + s * 48
            for bp in range(_GL):
                rvec = rbase + bp
                for l in range(_GL):
                    p = bp * _GL + l
                    for v in range(_DV):
                        cvec = jnp.full((16,), _GL * v + l, jnp.int32)
                        val = plsc.load_gather(strip_v, [rvec, cvec])
                        outb_v[p, 2 * v + ti_off, 0, pl.ds(c_off, 16)] = val
            return inner

        lax.fori_loop(0, 16, lane_loop, 0)
        for p in range(_NP):
            pltpu.sync_copy(
                outb_v.at[p],
                out_hbm.at[batch, p, th, pl.ds(t * 16, 16), pl.ds(hh, 1), :],
            )
        return carry

    lax.fori_loop(0, _NT, strip_loop, 0)


def kernel(inputs):
    A = inputs
    B, H, S1, S2 = A.shape
    a = A.reshape(B * H, S1, S2)
    mesh = plsc.VectorSubcoreMesh(
        core_axis_name="c", subcore_axis_name="s", num_cores=2, num_subcores=16
    )
    f = pl.kernel(
        _sc_body,
        mesh=mesh,
        compiler_params=pltpu.CompilerParams(
            use_tc_tiling_on_sc=False, needs_layout_passes=False
        ),
        out_type=jax.ShapeDtypeStruct((B, _NP, 2, 512, 8, 128), jnp.float32),
        scratch_types=[
            pltpu.VMEM((_S, _CW), jnp.float32),
            pltpu.VMEM((_NP, 16, 1, 128), jnp.float32),
        ],
    )
    out = f(a)
    o = out.transpose(0, 1, 2, 4, 3, 5).reshape(B, _GL, _GL, H, 65536)
    return o.transpose(0, 3, 4, 1, 2)


# speedup vs baseline: 6.1769x; 1.1235x over previous
import jax
import jax.numpy as jnp
from jax import lax
from jax.experimental import pallas as pl
from jax.experimental.pallas import tpu as pltpu
from jax.experimental.pallas import tpu_sc as plsc

_S = 768
_GL = 3
_NT = 32
_CW = 24
_DV = 8
_NP = 9


def _sc_body(a_hbm, out_hbm, strip0_v, strip1_v, outb_v, sem_in):
    wid = lax.axis_index("c") * 16 + lax.axis_index("s")
    batch = wid // 16
    h = wid - batch * 16
    th = h // 8
    hh = h - th * 8

    lane = lax.iota(jnp.int32, 16)
    r3 = lane * 3
    strips = (strip0_v, strip1_v)

    def src(t):
        return a_hbm.at[wid, :, pl.ds(t * _CW, _CW)]

    pltpu.async_copy(src(0), strip0_v, sem_in)

    def strip_pair_loop(t2, carry):
        for par in range(2):
            t = 2 * t2 + par
            cur = strips[par]
            nxt = strips[1 - par]

            @pl.when(t + 1 < _NT)
            def _prefetch():
                pltpu.async_copy(src(t + 1), nxt, sem_in)

            pltpu.make_async_copy(src(t), cur, sem_in).wait()

            def lane_loop(s, inner):
                ti_off = s >> 3
                c_off = (s & 7) * 16
                rbase = r3 + s * 48
                for bp in range(_GL):
                    rvec = rbase + bp
                    for l in range(_GL):
                        p = bp * _GL + l
                        for v in range(_DV):
                            cvec = jnp.full((16,), _GL * v + l, jnp.int32)
                            val = plsc.load_gather(cur, [rvec, cvec])
                            outb_v[p, 2 * v + ti_off, 0, pl.ds(c_off, 16)] = val
                return inner

            lax.fori_loop(0, 16, lane_loop, 0)
            for p in range(_NP):
                pltpu.sync_copy(
                    outb_v.at[p],
                    out_hbm.at[batch, p, th, pl.ds(t * 16, 16), pl.ds(hh, 1), :],
                )
        return carry

    lax.fori_loop(0, _NT // 2, strip_pair_loop, 0)


def kernel(inputs):
    A = inputs
    B, H, S1, S2 = A.shape
    a = A.reshape(B * H, S1, S2)
    mesh = plsc.VectorSubcoreMesh(
        core_axis_name="c", subcore_axis_name="s", num_cores=2, num_subcores=16
    )
    f = pl.kernel(
        _sc_body,
        mesh=mesh,
        compiler_params=pltpu.CompilerParams(
            use_tc_tiling_on_sc=False, needs_layout_passes=False
        ),
        out_type=jax.ShapeDtypeStruct((B, _NP, 2, 512, 8, 128), jnp.float32),
        scratch_types=[
            pltpu.VMEM((_S, _CW), jnp.float32),
            pltpu.VMEM((_S, _CW), jnp.float32),
            pltpu.VMEM((_NP, 16, 1, 128), jnp.float32),
            pltpu.SemaphoreType.DMA,
        ],
    )
    out = f(a)
    o = out.transpose(0, 1, 2, 4, 3, 5).reshape(B, _GL, _GL, H, 65536)
    return o.transpose(0, 3, 4, 1, 2)
